# Initial kernel scaffold; baseline (speedup 1.0000x reference)
import jax
import jax.numpy as jnp
from jax.experimental import pallas as pl


def kernel(x, edge_index, W0, as0, ad0, b0, W1, as1, ad1, b1, W2, as2, ad2, b2, Wf, bf):
    raise NotImplementedError("write your pallas kernel here")


# speedup vs baseline: 1.0470x; 1.0470x over previous
import functools
import jax
import jax.numpy as jnp
from jax import lax
from jax.experimental import pallas as pl
from jax.experimental.pallas import tpu as pltpu

_N = 10000
_NPAD = 10240
_E = 320000
_HEADS = 8
_OUT = 128
_HID = 1024
_MB = 512


def _mm_body(x_ref, w_ref, o_ref):
    o_ref[...] = jnp.dot(x_ref[...], w_ref[...],
                         preferred_element_type=jnp.float32)


def _mm(x, w):
    m, k = x.shape
    n = w.shape[1]
    grid = (m // _MB,)
    return pl.pallas_call(
        _mm_body,
        grid=grid,
        in_specs=[
            pl.BlockSpec((_MB, k), lambda i: (i, 0)),
            pl.BlockSpec((k, n), lambda i: (0, 0)),
        ],
        out_specs=pl.BlockSpec((_MB, n), lambda i: (i, 0)),
        out_shape=jax.ShapeDtypeStruct((m, n), jnp.float32),
    )(x, w)


def _attn_proj(a_src, a_dst):
    eye = jnp.eye(_HEADS, dtype=jnp.float32)
    a_s = (a_src[:, :, None] * eye[:, None, :]).reshape(_HID, _HEADS)
    a_d = (a_dst[:, :, None] * eye[:, None, :]).reshape(_HID, _HEADS)
    proj = jnp.concatenate([a_s, a_d], axis=1)
    return jnp.pad(proj, ((0, 0), (0, 128 - 2 * _HEADS)))


def _gat_layer(h_in, src, dst, w, a_src, a_dst, b):
    h = _mm(h_in, w)
    aa = _mm(h, _attn_proj(a_src, a_dst))
    asrc = aa[:_N, :_HEADS]
    adst = aa[:_N, _HEADS:2 * _HEADS]

    alpha = asrc[src] + adst[dst]
    alpha = jnp.where(alpha >= 0, alpha, 0.2 * alpha)
    amax = jax.ops.segment_max(alpha, dst, num_segments=_N)
    amax = jnp.where(jnp.isfinite(amax), amax, 0.0)
    e = jnp.exp(alpha - amax[dst])
    denom = jax.ops.segment_sum(e, dst, num_segments=_N) + 1e-16

    hv = h[:_N].reshape(_N, _HEADS, _OUT)
    msg = hv[src] * e[:, :, None]
    acc = jax.ops.segment_sum(msg, dst, num_segments=_N)
    out = acc / denom[:, :, None]
    return out.reshape(_N, _HID) + b[None, :]


def kernel(x, edge_index, W0, as0, ad0, b0, W1, as1, ad1, b1,
           W2, as2, ad2, b2, Wf, bf):
    loops = jnp.arange(_N, dtype=edge_index.dtype)
    ei = jnp.concatenate([edge_index, jnp.stack([loops, loops])], axis=1)
    src, dst = ei[0], ei[1]

    def pad(h):
        return jnp.pad(h, ((0, _NPAD - _N), (0, 0)))

    h = _gat_layer(pad(x), src, dst, W0, as0, ad0, b0)
    h = jax.nn.elu(h)
    h = _gat_layer(pad(h), src, dst, W1, as1, ad1, b1)
    h = jax.nn.elu(h)
    h = _gat_layer(pad(h), src, dst, W2, as2, ad2, b2)
    out = _mm(pad(h), Wf)[:_N] + bf[None, :]
    return out


# speedup vs baseline: 3.0492x; 2.9123x over previous
import functools
import jax
import jax.numpy as jnp
from jax import lax
from jax.experimental import pallas as pl
from jax.experimental.pallas import tpu as pltpu
from jax.experimental.pallas import tpu_sc as plsc

_N = 10000
_NPAD = 10240
_NSC = 10112
_E = 320000
_EP = 330240
_PER_TILE = 10320
_CH = 16
_NCHUNK = _PER_TILE // _CH
_HEADS = 8
_OUT = 128
_HID = 1024
_MB = 512
_ROWS_PER_SUB = _NSC // 16

_mesh = plsc.VectorSubcoreMesh(core_axis_name="c", subcore_axis_name="s")


def _mm_body(x_ref, w_ref, o_ref):
    o_ref[...] = jnp.dot(x_ref[...], w_ref[...],
                         preferred_element_type=jnp.float32)


def _mm(x, w):
    m, k = x.shape
    n = w.shape[1]
    return pl.pallas_call(
        _mm_body,
        grid=(m // _MB,),
        in_specs=[
            pl.BlockSpec((_MB, k), lambda i: (i, 0)),
            pl.BlockSpec((k, n), lambda i: (0, 0)),
        ],
        out_specs=pl.BlockSpec((_MB, n), lambda i: (i, 0)),
        out_shape=jax.ShapeDtypeStruct((m, n), jnp.float32),
    )(x, w)


def _attn_proj(a_src, a_dst):
    eye = jnp.eye(_HEADS, dtype=jnp.float32)
    a_s = (a_src[:, :, None] * eye[:, None, :]).reshape(_HID, _HEADS)
    a_d = (a_dst[:, :, None] * eye[:, None, :]).reshape(_HID, _HEADS)
    proj = jnp.concatenate([a_s, a_d], axis=1)
    return jnp.pad(proj, ((0, 0), (0, 128 - 2 * _HEADS)))


@functools.partial(
    pl.kernel, mesh=_mesh,
    out_type=[
        jax.ShapeDtypeStruct((_EP, 16), jnp.float32),
        jax.ShapeDtypeStruct((2, _NSC, 128), jnp.float32),
    ],
    scratch_types=[
        pltpu.VMEM((_CH,), jnp.int32),
        pltpu.VMEM((_CH,), jnp.int32),
        pltpu.VMEM((_CH, 128), jnp.float32),
        pltpu.VMEM((_CH, 128), jnp.float32),
        pltpu.VMEM((_CH, 16), jnp.float32),
        pltpu.VMEM((_CH, 128), jnp.float32),
        pltpu.VMEM_SHARED((_NSC, 128), jnp.float32),
        pltpu.SemaphoreType.DMA,
        pltpu.SemaphoreType.DMA,
    ],
)
def _sc_edge_weights(tt_h, src_h, dst_h, z128_h,
                     e_out, den_out,
                     src_v, dst_v, ts_v, td_v, e_v, e128_v,
                     den_sh, sem0, sem1):
    cid = lax.axis_index("c")
    sid = lax.axis_index("s")
    wid = sid * 2 + cid
    r0 = sid * _ROWS_PER_SUB
    pltpu.sync_copy(z128_h.at[pl.ds(r0, _ROWS_PER_SUB)],
                    den_sh.at[pl.ds(r0, _ROWS_PER_SUB)])
    zero16 = jnp.full((16,), 0.0, jnp.float32)
    for i in range(_CH):
        for j in range(8):
            e128_v[i, pl.ds(j * 16, 16)] = zero16
    plsc.subcore_barrier()

    lane = lax.broadcasted_iota(jnp.int32, (16,), 0)
    lane8 = jnp.full((16,), _HEADS, jnp.int32)
    slope = jnp.full((16,), 0.2, jnp.float32)
    msk = lane < lane8

    def body(c, carry):
        base = wid * _PER_TILE + c * _CH
        pltpu.sync_copy(src_h.at[pl.ds(base, _CH)], src_v)
        pltpu.sync_copy(dst_h.at[pl.ds(base, _CH)], dst_v)
        pltpu.async_copy(tt_h.at[src_v], ts_v, sem0).wait()
        pltpu.async_copy(tt_h.at[dst_v], td_v, sem1).wait()
        for i in range(_CH):
            a = ts_v[i, pl.ds(0, 16)] + td_v[i, pl.ds(16, 16)]
            l = jnp.maximum(a, zero16) + slope * jnp.minimum(a, zero16)
            e = jnp.where(msk, jnp.exp(l - td_v[i, pl.ds(32, 16)]), zero16)
            e_v[i] = e
            e128_v[i, pl.ds(0, 16)] = e
        pltpu.sync_copy(e_v, e_out.at[pl.ds(base, _CH)])
        pltpu.sync_copy(e128_v, den_sh.at[dst_v], add=True)
        return carry

    lax.fori_loop(0, _NCHUNK, body, 0)
    plsc.subcore_barrier()
    pltpu.sync_copy(den_sh.at[pl.ds(r0, _ROWS_PER_SUB)],
                    den_out.at[cid].at[pl.ds(r0, _ROWS_PER_SUB)])


@functools.partial(
    pl.kernel, mesh=_mesh,
    out_type=jax.ShapeDtypeStruct((2, _HEADS, _NSC, _OUT), jnp.float32),
    scratch_types=[
        pltpu.VMEM((_CH,), jnp.int32),
        pltpu.VMEM((_CH,), jnp.int32),
        pltpu.VMEM((_CH, 16), jnp.float32),
        pltpu.VMEM((_CH, _OUT), jnp.float32),
        pltpu.VMEM_SHARED((_NSC, _OUT), jnp.float32),
        pltpu.SemaphoreType.DMA,
    ],
)
def _sc_aggregate(h0, h1, h2, h3, h4, h5, h6, h7,
                  e_h, src_h, dst_h, z128_h,
                  acc_out,
                  src_v, dst_v, e_v, rows_v, acc_sh, sem):
    cid = lax.axis_index("c")
    sid = lax.axis_index("s")
    wid = sid * 2 + cid
    r0 = sid * _ROWS_PER_SUB
    htabs = (h0, h1, h2, h3, h4, h5, h6, h7)
    for hd in range(_HEADS):
        pltpu.sync_copy(z128_h.at[pl.ds(r0, _ROWS_PER_SUB)],
                        acc_sh.at[pl.ds(r0, _ROWS_PER_SUB)])
        plsc.subcore_barrier()

        def body(c, carry):
            base = wid * _PER_TILE + c * _CH
            pltpu.sync_copy(src_h.at[pl.ds(base, _CH)], src_v)
            pltpu.sync_copy(dst_h.at[pl.ds(base, _CH)], dst_v)
            pltpu.sync_copy(e_h.at[pl.ds(base, _CH)], e_v)
            pltpu.async_copy(htabs[hd].at[src_v], rows_v, sem).wait()
            for i in range(_CH):
                s = lax.gather(
                    e_v[i], jnp.full((16, 1), hd, jnp.int32),
                    lax.GatherDimensionNumbers(
                        offset_dims=(), collapsed_slice_dims=(0,),
                        start_index_map=(0,)),
                    (1,), mode=lax.GatherScatterMode.PROMISE_IN_BOUNDS)
                for j in range(_OUT // 16):
                    sl = pl.ds(j * 16, 16)
                    rows_v[i, sl] = rows_v[i, sl] * s
            pltpu.sync_copy(rows_v, acc_sh.at[dst_v], add=True)
            return carry

        lax.fori_loop(0, _NCHUNK, body, 0)
        plsc.subcore_barrier()
        pltpu.sync_copy(acc_sh.at[pl.ds(r0, _ROWS_PER_SUB)],
                        acc_out.at[cid].at[hd].at[pl.ds(r0, _ROWS_PER_SUB)])


def _pad_rows(a, rows):
    return jnp.pad(a, ((0, rows - a.shape[0]), (0, 0)))


def _gat_layer(h_in, src, dst, z128, w, a_src, a_dst, b):
    h = _mm(h_in, w)
    aa = _mm(h, _attn_proj(a_src, a_dst))
    asrc = aa[:_N, :_HEADS]
    adst = aa[:_N, _HEADS:2 * _HEADS]
    m = jnp.max(asrc, axis=0)
    z = m[None, :] + adst
    ub = jnp.where(z >= 0, z, 0.2 * z)

    z8 = jnp.zeros((_N, 8), jnp.float32)
    tt = jnp.concatenate(
        [asrc, z8, adst, z8, ub, jnp.zeros((_N, 88), jnp.float32)], axis=1)
    tt = jnp.pad(tt, ((0, _NSC - _N), (0, 0)))

    e_rows, den = _sc_edge_weights(tt, src, dst, z128)
    denom = (den[0] + den[1])[:_N, :_HEADS] + 1e-16

    ht = h[:_NSC].reshape(_NSC, _HEADS, _OUT).transpose(1, 0, 2)
    acc = _sc_aggregate(ht[0], ht[1], ht[2], ht[3],
                        ht[4], ht[5], ht[6], ht[7],
                        e_rows, src, dst, z128)
    agg = (acc[0] + acc[1]).transpose(1, 0, 2)[:_N]
    out = agg / denom[:, :, None]
    return out.reshape(_N, _HID) + b[None, :]


def kernel(x, edge_index, W0, as0, ad0, b0, W1, as1, ad1, b1,
           W2, as2, ad2, b2, Wf, bf):
    loops = jnp.arange(_N, dtype=edge_index.dtype)
    dummy = jnp.full((_EP - _E - _N,), _N, dtype=edge_index.dtype)
    src = jnp.concatenate([edge_index[0], loops, dummy])
    dst = jnp.concatenate([edge_index[1], loops, dummy])
    z128 = jnp.zeros((_NSC, _OUT), jnp.float32)

    h = _gat_layer(_pad_rows(x, _NPAD), src, dst, z128,
                   W0, as0, ad0, b0)
    h = jax.nn.elu(h)
    h = _gat_layer(_pad_rows(h, _NPAD), src, dst, z128,
                   W1, as1, ad1, b1)
    h = jax.nn.elu(h)
    h = _gat_layer(_pad_rows(h, _NPAD), src, dst, z128,
                   W2, as2, ad2, b2)
    return _mm(_pad_rows(h, _NPAD), Wf)[:_N] + bf[None, :]
